# baseline (speedup 1.0000x reference)
import jax
import jax.numpy as jnp
from jax.experimental import pallas as pl
from jax.experimental.pallas import tpu as pltpu

_PRED_T = 0.3
_TRUE_T = 0.5
_PW = 0.05
_HB = 56


def _bbox_body(p_ref, e_ref, out_ref, yb, psum_ref):
    b = pl.program_id(0)
    h = pl.program_id(1)
    nb = pl.num_programs(0)
    nh = pl.num_programs(1)
    HB, WC = p_ref.shape[1], p_ref.shape[2]
    H = HB * nh
    C = 96
    W = WC // C
    f32 = jnp.float32

    p = p_ref[0]
    e = e_ref[0]
    prow = jnp.max(p, axis=1)
    erow = jnp.max(e, axis=1)
    pcol = jnp.max(p, axis=0)
    ecol = jnp.max(e, axis=0)

    hidx = jax.lax.broadcasted_iota(jnp.int32, (HB,), 0).astype(f32) + jnp.float32(
        h * HB
    )
    widx = jnp.floor(
        jax.lax.broadcasted_iota(jnp.int32, (WC,), 0).astype(f32) * (1.0 / C)
    )

    fH = jnp.float32(H)
    fW = jnp.float32(W)
    first = h == 0
    pymin = jnp.min(jnp.where(prow > _PRED_T, hidx, fH))
    pymax = jnp.max(jnp.where(prow > _PRED_T, hidx, -1.0))
    pxmin = jnp.min(jnp.where(pcol > _PRED_T, widx, fW))
    pxmax = jnp.max(jnp.where(pcol > _PRED_T, widx, -1.0))
    tymin = jnp.min(jnp.where(erow > _TRUE_T, hidx, fH))
    tymax = jnp.max(jnp.where(erow > _TRUE_T, hidx, -1.0))
    txmin = jnp.min(jnp.where(ecol > _TRUE_T, widx, fW))
    txmax = jnp.max(jnp.where(ecol > _TRUE_T, widx, -1.0))

    yb[0] = jnp.minimum(jnp.where(first, fH, yb[0]), pymin)
    yb[1] = jnp.maximum(jnp.where(first, -1.0, yb[1]), pymax)
    yb[2] = jnp.minimum(jnp.where(first, fW, yb[2]), pxmin)
    yb[3] = jnp.maximum(jnp.where(first, -1.0, yb[3]), pxmax)
    yb[4] = jnp.minimum(jnp.where(first, fH, yb[4]), tymin)
    yb[5] = jnp.maximum(jnp.where(first, -1.0, yb[5]), tymax)
    yb[6] = jnp.minimum(jnp.where(first, fW, yb[6]), txmin)
    yb[7] = jnp.maximum(jnp.where(first, -1.0, yb[7]), txmax)

    @pl.when(h == nh - 1)
    def _tail():
        def vec(s):
            return jnp.full((1, 128), s, f32)

        py1, py2, px1, px2 = yb[0], yb[1], yb[2], yb[3]
        ty1, ty2, tx1, tx2 = yb[4], yb[5], yb[6], yb[7]
        pa = vec((py2 - py1 + 1.0) * (px2 - px1 + 1.0))
        ta = vec((ty2 - ty1 + 1.0) * (tx2 - tx1 + 1.0))
        area_pen = jnp.maximum(pa - ta, 0.0) / (ta + 1.0)
        cy = vec(py1 + py2) * 0.5 - vec(ty1 + ty2) * 0.5
        cx = vec(px1 + px2) * 0.5 - vec(tx1 + tx2) * 0.5
        center = jnp.sqrt(cy * cy + cx * cx) * (1.0 / 20.0)
        valid = jnp.full((1, 128), (py2 >= 0.0) & (ty2 >= 0.0), jnp.bool_)
        pen = jnp.where(valid, area_pen + center, 1.0)
        prev = jnp.where(b == 0, jnp.zeros_like(pen), psum_ref[...])
        tot = prev + pen
        psum_ref[...] = tot

        @pl.when(b == nb - 1)
        def _():
            out_ref[...] = tot * (_PW / nb)


def kernel(prediction_probs, expected_onehot):
    B, H, W, C = prediction_probs.shape
    pf = prediction_probs.reshape(B, H, W * C)
    ef = expected_onehot.reshape(B, H, W * C)
    nh = H // _HB
    out = pl.pallas_call(
        _bbox_body,
        grid=(B, nh),
        in_specs=[
            pl.BlockSpec((1, _HB, W * C), lambda b, h: (b, h, 0)),
            pl.BlockSpec((1, _HB, W * C), lambda b, h: (b, h, 0)),
        ],
        out_specs=pl.BlockSpec((1, 128), lambda b, h: (0, 0)),
        out_shape=jax.ShapeDtypeStruct((1, 128), jnp.float32),
        scratch_shapes=[
            pltpu.SMEM((8,), jnp.float32),
            pltpu.VMEM((1, 128), jnp.float32),
        ],
    )(pf, ef)
    return out[0, 0]


# speedup vs baseline: 4.2142x; 4.2142x over previous
import jax
import jax.numpy as jnp
from jax.experimental import pallas as pl
from jax.experimental.pallas import tpu as pltpu

_PRED_T = 0.3
_TRUE_T = 0.5
_PW = 0.05
_HB = 56


def _bbox_body(p_ref, e_ref, out_ref, yb, psum_ref):
    b = pl.program_id(0)
    h = pl.program_id(1)
    nb = pl.num_programs(0)
    nh = pl.num_programs(1)
    HB, W = p_ref.shape[1], p_ref.shape[2]
    H = HB * nh
    f32 = jnp.float32

    p = p_ref[0]
    e = e_ref[0]
    prow = jnp.max(jnp.max(p, axis=1), axis=1)
    erow = jnp.max(jnp.max(e, axis=1), axis=1)
    pcol = jnp.max(jnp.max(p, axis=0), axis=1)
    ecol = jnp.max(jnp.max(e, axis=0), axis=1)

    hidx = jax.lax.broadcasted_iota(jnp.int32, (HB,), 0).astype(f32) + jnp.float32(
        h * HB
    )
    widx = jax.lax.broadcasted_iota(jnp.int32, (W,), 0).astype(f32)

    fH = jnp.float32(H)
    fW = jnp.float32(W)
    first = h == 0
    pymin = jnp.min(jnp.where(prow > _PRED_T, hidx, fH))
    pymax = jnp.max(jnp.where(prow > _PRED_T, hidx, -1.0))
    pxmin = jnp.min(jnp.where(pcol > _PRED_T, widx, fW))
    pxmax = jnp.max(jnp.where(pcol > _PRED_T, widx, -1.0))
    tymin = jnp.min(jnp.where(erow > _TRUE_T, hidx, fH))
    tymax = jnp.max(jnp.where(erow > _TRUE_T, hidx, -1.0))
    txmin = jnp.min(jnp.where(ecol > _TRUE_T, widx, fW))
    txmax = jnp.max(jnp.where(ecol > _TRUE_T, widx, -1.0))

    yb[0] = jnp.minimum(jnp.where(first, fH, yb[0]), pymin)
    yb[1] = jnp.maximum(jnp.where(first, -1.0, yb[1]), pymax)
    yb[2] = jnp.minimum(jnp.where(first, fW, yb[2]), pxmin)
    yb[3] = jnp.maximum(jnp.where(first, -1.0, yb[3]), pxmax)
    yb[4] = jnp.minimum(jnp.where(first, fH, yb[4]), tymin)
    yb[5] = jnp.maximum(jnp.where(first, -1.0, yb[5]), tymax)
    yb[6] = jnp.minimum(jnp.where(first, fW, yb[6]), txmin)
    yb[7] = jnp.maximum(jnp.where(first, -1.0, yb[7]), txmax)

    @pl.when(h == nh - 1)
    def _tail():
        def vec(s):
            return jnp.full((1, 128), s, f32)

        py1, py2, px1, px2 = yb[0], yb[1], yb[2], yb[3]
        ty1, ty2, tx1, tx2 = yb[4], yb[5], yb[6], yb[7]
        pa = vec((py2 - py1 + 1.0) * (px2 - px1 + 1.0))
        ta = vec((ty2 - ty1 + 1.0) * (tx2 - tx1 + 1.0))
        area_pen = jnp.maximum(pa - ta, 0.0) / (ta + 1.0)
        cy = vec(py1 + py2) * 0.5 - vec(ty1 + ty2) * 0.5
        cx = vec(px1 + px2) * 0.5 - vec(tx1 + tx2) * 0.5
        center = jnp.sqrt(cy * cy + cx * cx) * (1.0 / 20.0)
        valid = jnp.full((1, 128), (py2 >= 0.0) & (ty2 >= 0.0), jnp.bool_)
        pen = jnp.where(valid, area_pen + center, 1.0)
        prev = jnp.where(b == 0, jnp.zeros_like(pen), psum_ref[...])
        tot = prev + pen
        psum_ref[...] = tot

        @pl.when(b == nb - 1)
        def _():
            out_ref[...] = tot * (_PW / nb)


def kernel(prediction_probs, expected_onehot):
    B, H, W, C = prediction_probs.shape
    nh = H // _HB
    out = pl.pallas_call(
        _bbox_body,
        grid=(B, nh),
        in_specs=[
            pl.BlockSpec((1, _HB, W, C), lambda b, h: (b, h, 0, 0)),
            pl.BlockSpec((1, _HB, W, C), lambda b, h: (b, h, 0, 0)),
        ],
        out_specs=pl.BlockSpec((1, 128), lambda b, h: (0, 0)),
        out_shape=jax.ShapeDtypeStruct((1, 128), jnp.float32),
        scratch_shapes=[
            pltpu.SMEM((8,), jnp.float32),
            pltpu.VMEM((1, 128), jnp.float32),
        ],
    )(prediction_probs, expected_onehot)
    return out[0, 0]


# speedup vs baseline: 17.3561x; 4.1185x over previous
import jax
import jax.numpy as jnp
from jax.experimental import pallas as pl
from jax.experimental.pallas import tpu as pltpu

_PRED_T = 0.3
_TRUE_T = 0.5
_PW = 0.05
_HB = 56


def _bbox_body(p_ref, e_ref, out_ref, yb, psum_ref):
    b = pl.program_id(0)
    h = pl.program_id(1)
    nb = pl.num_programs(0)
    nh = pl.num_programs(1)
    HB, W = p_ref.shape[1], p_ref.shape[3]
    H = HB * nh
    f32 = jnp.float32

    p = p_ref[0]
    e = e_ref[0]
    pm = jnp.max(p, axis=1)
    em = jnp.max(e, axis=1)
    prow = jnp.max(pm, axis=1)
    erow = jnp.max(em, axis=1)
    pcol = jnp.max(pm, axis=0)
    ecol = jnp.max(em, axis=0)

    hidx = jax.lax.broadcasted_iota(jnp.int32, (HB,), 0).astype(f32) + jnp.float32(
        h * HB
    )
    widx = jax.lax.broadcasted_iota(jnp.int32, (W,), 0).astype(f32)

    fH = jnp.float32(H)
    fW = jnp.float32(W)
    first = h == 0
    pymin = jnp.min(jnp.where(prow > _PRED_T, hidx, fH))
    pymax = jnp.max(jnp.where(prow > _PRED_T, hidx, -1.0))
    pxmin = jnp.min(jnp.where(pcol > _PRED_T, widx, fW))
    pxmax = jnp.max(jnp.where(pcol > _PRED_T, widx, -1.0))
    tymin = jnp.min(jnp.where(erow > _TRUE_T, hidx, fH))
    tymax = jnp.max(jnp.where(erow > _TRUE_T, hidx, -1.0))
    txmin = jnp.min(jnp.where(ecol > _TRUE_T, widx, fW))
    txmax = jnp.max(jnp.where(ecol > _TRUE_T, widx, -1.0))

    yb[0] = jnp.minimum(jnp.where(first, fH, yb[0]), pymin)
    yb[1] = jnp.maximum(jnp.where(first, -1.0, yb[1]), pymax)
    yb[2] = jnp.minimum(jnp.where(first, fW, yb[2]), pxmin)
    yb[3] = jnp.maximum(jnp.where(first, -1.0, yb[3]), pxmax)
    yb[4] = jnp.minimum(jnp.where(first, fH, yb[4]), tymin)
    yb[5] = jnp.maximum(jnp.where(first, -1.0, yb[5]), tymax)
    yb[6] = jnp.minimum(jnp.where(first, fW, yb[6]), txmin)
    yb[7] = jnp.maximum(jnp.where(first, -1.0, yb[7]), txmax)

    @pl.when(h == nh - 1)
    def _tail():
        def vec(s):
            return jnp.full((1, 128), s, f32)

        py1, py2, px1, px2 = yb[0], yb[1], yb[2], yb[3]
        ty1, ty2, tx1, tx2 = yb[4], yb[5], yb[6], yb[7]
        pa = vec((py2 - py1 + 1.0) * (px2 - px1 + 1.0))
        ta = vec((ty2 - ty1 + 1.0) * (tx2 - tx1 + 1.0))
        area_pen = jnp.maximum(pa - ta, 0.0) / (ta + 1.0)
        cy = vec(py1 + py2) * 0.5 - vec(ty1 + ty2) * 0.5
        cx = vec(px1 + px2) * 0.5 - vec(tx1 + tx2) * 0.5
        center = jnp.sqrt(cy * cy + cx * cx) * (1.0 / 20.0)
        valid = jnp.full((1, 128), (py2 >= 0.0) & (ty2 >= 0.0), jnp.bool_)
        pen = jnp.where(valid, area_pen + center, 1.0)
        prev = jnp.where(b == 0, jnp.zeros_like(pen), psum_ref[...])
        tot = prev + pen
        psum_ref[...] = tot

        @pl.when(b == nb - 1)
        def _():
            out_ref[...] = tot * (_PW / nb)


def kernel(prediction_probs, expected_onehot):
    B, H, W, C = prediction_probs.shape
    pt = prediction_probs.transpose(0, 1, 3, 2)
    et = expected_onehot.transpose(0, 1, 3, 2)
    nh = H // _HB
    out = pl.pallas_call(
        _bbox_body,
        grid=(B, nh),
        in_specs=[
            pl.BlockSpec((1, _HB, C, W), lambda b, h: (b, h, 0, 0)),
            pl.BlockSpec((1, _HB, C, W), lambda b, h: (b, h, 0, 0)),
        ],
        out_specs=pl.BlockSpec((1, 128), lambda b, h: (0, 0)),
        out_shape=jax.ShapeDtypeStruct((1, 128), jnp.float32),
        scratch_shapes=[
            pltpu.SMEM((8,), jnp.float32),
            pltpu.VMEM((1, 128), jnp.float32),
        ],
    )(pt, et)
    return out[0, 0]
